# baseline (speedup 1.0000x reference)
import functools

import jax
import jax.numpy as jnp
from jax import lax
from jax.experimental import pallas as pl
from jax.experimental.pallas import tpu as pltpu
from jax.experimental.pallas import tpu_sc as plsc

B = 8
N_IN = 196608
N_OUT = 49152
K = 4
C_IN = 16
FILTERS = 32
D = B * C_IN
KW = K * D
NF = B * FILTERS

NSLICE = 2
P_S = N_OUT // NSLICE
NK_S = P_S * K

NC = 2
NS = 16
NW = NC * NS
PER_W = NK_S // NW
IDX_MINOR = 128
IDX_ROWS = PER_W // IDX_MINOR
CHUNK_ROWS = 2
CHUNK = CHUNK_ROWS * IDX_MINOR
NCHUNK = IDX_ROWS // CHUNK_ROWS
NBUF = 3
NROUND = NCHUNK // NBUF

_sc_mesh = plsc.VectorSubcoreMesh(core_axis_name="c", subcore_axis_name="s")


@functools.partial(
    pl.kernel,
    out_type=jax.ShapeDtypeStruct((NK_S, D), jnp.float32),
    mesh=_sc_mesh,
    scratch_types=[
        pltpu.VMEM((IDX_ROWS, IDX_MINOR), jnp.int32),
        pltpu.VMEM((NBUF, CHUNK, D), jnp.float32),
        pltpu.SemaphoreType.DMA,
        pltpu.SemaphoreType.DMA,
        pltpu.SemaphoreType.DMA,
        pltpu.SemaphoreType.DMA,
        pltpu.SemaphoreType.DMA,
        pltpu.SemaphoreType.DMA,
    ],
)
def _sc_gather(xt_hbm, idx_hbm, z_hbm, idx_v, rows_v, g0, g1, g2, w0, w1, w2):
    gsem = (g0, g1, g2)
    wsem = (w0, w1, w2)
    wid = lax.axis_index("s") * NC + lax.axis_index("c")
    base = wid * PER_W
    pltpu.sync_copy(idx_hbm.at[wid], idx_v)

    def issue_gather(ch, p):
        for j in range(CHUNK_ROWS):
            pltpu.async_copy(
                xt_hbm.at[idx_v.at[ch * CHUNK_ROWS + j]],
                rows_v.at[p].at[pl.ds(j * IDX_MINOR, IDX_MINOR)],
                gsem[p],
            )

    def wait_gather(p):
        pltpu.make_async_copy(z_hbm.at[pl.ds(0, CHUNK)], rows_v.at[p], gsem[p]).wait()

    def issue_wb(ch, p):
        pltpu.async_copy(rows_v.at[p], z_hbm.at[pl.ds(base + ch * CHUNK, CHUNK)], wsem[p])

    def wait_wb(p):
        pltpu.make_async_copy(z_hbm.at[pl.ds(0, CHUNK)], rows_v.at[p], wsem[p]).wait()

    issue_gather(0, 0)
    issue_gather(1, 1)

    def round_(r, carry):
        for p in range(NBUF):
            ch = r * NBUF + p
            if p == 0:
                @pl.when(r >= 1)
                def _():
                    wait_wb(2)
                issue_gather(ch + 2, 2)
            else:
                @pl.when(ch + 2 < NCHUNK)
                def _():
                    wait_wb(p - 1)
                    issue_gather(ch + 2, p - 1)
            wait_gather(p)
            issue_wb(ch, p)
        return carry

    lax.fori_loop(0, NROUND, round_, 0)
    for p in range(NBUF):
        wait_wb(p)


TILE = 4096
NT_S = P_S // TILE

TN = 1024


def _tr_body(x_ref, o_ref):
    o_ref[...] = x_ref[...].T


_transpose = pl.pallas_call(
    _tr_body,
    grid=(N_IN // TN,),
    in_specs=[pl.BlockSpec((D, TN), lambda i: (0, i))],
    out_specs=pl.BlockSpec((TN, D), lambda i: (i, 0)),
    out_shape=jax.ShapeDtypeStruct((N_IN, D), jnp.float32),
)


def _mm_body(z0, z1, z2, z3, w0, w1, w2, w3, bias_ref, o_ref):
    acc = bias_ref[...]
    for zk, wk in zip((z0, z1, z2, z3), (w0, w1, w2, w3)):
        acc = acc + jax.lax.dot_general(
            wk[...], zk[...],
            dimension_numbers=(((1,), (1,)), ((), ())),
            preferred_element_type=jnp.float32,
        )
    o_ref[...] = acc


def _mm_body_chained(z0, z1, z2, z3, w0, w1, w2, w3, bias_ref, y_ref, o_ref):
    _mm_body(z0, z1, z2, z3, w0, w1, w2, w3, bias_ref, o_ref)


def _z_spec(k):
    return pl.BlockSpec((TILE, D), lambda i, k=k: (k * NT_S + i, 0))


def _make_matmul(s, chained):
    in_specs = [
        _z_spec(0), _z_spec(1), _z_spec(2), _z_spec(3),
        pl.BlockSpec((NF, D), lambda i: (0, 0)),
        pl.BlockSpec((NF, D), lambda i: (0, 0)),
        pl.BlockSpec((NF, D), lambda i: (0, 0)),
        pl.BlockSpec((NF, D), lambda i: (0, 0)),
        pl.BlockSpec((NF, 1), lambda i: (0, 0)),
    ]
    kwargs = {}
    if chained:
        in_specs.append(pl.BlockSpec((8, 128), lambda i: (0, 0)))
        kwargs["input_output_aliases"] = {9: 0}
    return pl.pallas_call(
        _mm_body_chained if chained else _mm_body,
        grid=(NT_S,),
        in_specs=in_specs,
        out_specs=pl.BlockSpec((NF, TILE), lambda i, s=s: (0, s * NT_S + i)),
        out_shape=jax.ShapeDtypeStruct((NF, N_OUT), jnp.float32),
        **kwargs,
    )


_matmul0 = _make_matmul(0, chained=False)
_matmul1 = _make_matmul(1, chained=True)


def kernel(x, connection_indices, kernel, bias):
    xt = _transpose(x.transpose(0, 2, 1).reshape(D, N_IN))
    idx_t = connection_indices.astype(jnp.int32).T
    w_big = (
        jnp.eye(B, dtype=jnp.float32)[None, :, None, :, None]
        * kernel[:, None, :, None, :]
    ).reshape(KW, NF)
    w_t = w_big.T
    ws = (w_t[:, :D], w_t[:, D:2 * D], w_t[:, 2 * D:3 * D], w_t[:, 3 * D:])
    bias_t = jnp.tile(bias, B).reshape(NF, 1)

    z0 = _sc_gather(xt, idx_t[:, :P_S].reshape(NW, IDX_ROWS, IDX_MINOR))
    z1 = _sc_gather(xt, idx_t[:, P_S:].reshape(NW, IDX_ROWS, IDX_MINOR))
    y = _matmul0(z0, z0, z0, z0, *ws, bias_t)
    y = _matmul1(z1, z1, z1, z1, *ws, bias_t, y)
    return y.reshape(B, FILTERS, N_OUT).transpose(0, 2, 1)


# speedup vs baseline: 1.2600x; 1.2600x over previous
import functools

import jax
import jax.numpy as jnp
from jax import lax
from jax.experimental import pallas as pl
from jax.experimental.pallas import tpu as pltpu
from jax.experimental.pallas import tpu_sc as plsc

B = 8
N_IN = 196608
N_OUT = 49152
K = 4
C_IN = 16
FILTERS = 32
D = B * C_IN
KW = K * D
NF = B * FILTERS

NSLICE = 4
P_S = N_OUT // NSLICE
NK_S = P_S * K

NC = 2
NS = 16
NW = NC * NS
PER_W = NK_S // NW
IDX_MINOR = 128
IDX_ROWS = PER_W // IDX_MINOR
CHUNK_ROWS = 2
CHUNK = CHUNK_ROWS * IDX_MINOR
NCHUNK = IDX_ROWS // CHUNK_ROWS
NBUF = 3
NROUND = NCHUNK // NBUF

_sc_mesh = plsc.VectorSubcoreMesh(core_axis_name="c", subcore_axis_name="s")


@functools.partial(
    pl.kernel,
    out_type=jax.ShapeDtypeStruct((NK_S, D), jnp.float32),
    mesh=_sc_mesh,
    scratch_types=[
        pltpu.VMEM((IDX_ROWS, IDX_MINOR), jnp.int32),
        pltpu.VMEM((NBUF, CHUNK, D), jnp.float32),
        pltpu.SemaphoreType.DMA,
        pltpu.SemaphoreType.DMA,
        pltpu.SemaphoreType.DMA,
        pltpu.SemaphoreType.DMA,
        pltpu.SemaphoreType.DMA,
        pltpu.SemaphoreType.DMA,
    ],
)
def _sc_gather(xt_hbm, idx_hbm, z_hbm, idx_v, rows_v, g0, g1, g2, w0, w1, w2):
    gsem = (g0, g1, g2)
    wsem = (w0, w1, w2)
    wid = lax.axis_index("s") * NC + lax.axis_index("c")
    base = wid * PER_W
    pltpu.sync_copy(idx_hbm.at[wid], idx_v)

    def issue_gather(ch, p):
        for j in range(CHUNK_ROWS):
            pltpu.async_copy(
                xt_hbm.at[idx_v.at[ch * CHUNK_ROWS + j]],
                rows_v.at[p].at[pl.ds(j * IDX_MINOR, IDX_MINOR)],
                gsem[p],
            )

    def wait_gather(p):
        pltpu.make_async_copy(z_hbm.at[pl.ds(0, CHUNK)], rows_v.at[p], gsem[p]).wait()

    def issue_wb(ch, p):
        pltpu.async_copy(rows_v.at[p], z_hbm.at[pl.ds(base + ch * CHUNK, CHUNK)], wsem[p])

    def wait_wb(p):
        pltpu.make_async_copy(z_hbm.at[pl.ds(0, CHUNK)], rows_v.at[p], wsem[p]).wait()

    issue_gather(0, 0)
    issue_gather(1, 1)

    def round_(r, carry):
        for p in range(NBUF):
            ch = r * NBUF + p
            if p == 0:
                @pl.when(r >= 1)
                def _():
                    wait_wb(2)
                issue_gather(ch + 2, 2)
            else:
                @pl.when(ch + 2 < NCHUNK)
                def _():
                    wait_wb(p - 1)
                    issue_gather(ch + 2, p - 1)
            wait_gather(p)
            issue_wb(ch, p)
        return carry

    lax.fori_loop(0, NROUND, round_, 0)
    for p in range(NBUF):
        wait_wb(p)


TILE = 4096
NT_S = P_S // TILE


def _mm_body(z0, z1, z2, z3, w0, w1, w2, w3, bias_ref, o_ref):
    acc = bias_ref[...]
    for zk, wk in zip((z0, z1, z2, z3), (w0, w1, w2, w3)):
        acc = acc + jax.lax.dot_general(
            wk[...], zk[...],
            dimension_numbers=(((1,), (1,)), ((), ())),
            preferred_element_type=jnp.float32,
        )
    o_ref[...] = acc


def _mm_body_chained(z0, z1, z2, z3, w0, w1, w2, w3, bias_ref, y_ref, o_ref):
    _mm_body(z0, z1, z2, z3, w0, w1, w2, w3, bias_ref, o_ref)


def _z_spec(k):
    return pl.BlockSpec((TILE, D), lambda i, k=k: (k * NT_S + i, 0))


def _make_matmul(s, chained):
    in_specs = [
        _z_spec(0), _z_spec(1), _z_spec(2), _z_spec(3),
        pl.BlockSpec((NF, D), lambda i: (0, 0)),
        pl.BlockSpec((NF, D), lambda i: (0, 0)),
        pl.BlockSpec((NF, D), lambda i: (0, 0)),
        pl.BlockSpec((NF, D), lambda i: (0, 0)),
        pl.BlockSpec((NF, 1), lambda i: (0, 0)),
    ]
    kwargs = {}
    if chained:
        in_specs.append(pl.BlockSpec((8, 128), lambda i: (0, 0)))
        kwargs["input_output_aliases"] = {9: 0}
    return pl.pallas_call(
        _mm_body_chained if chained else _mm_body,
        grid=(NT_S,),
        in_specs=in_specs,
        out_specs=pl.BlockSpec((NF, TILE), lambda i, s=s: (0, s * NT_S + i)),
        out_shape=jax.ShapeDtypeStruct((NF, N_OUT), jnp.float32),
        **kwargs,
    )


_matmuls = [_make_matmul(s, chained=(s > 0)) for s in range(NSLICE)]


def kernel(x, connection_indices, kernel, bias):
    xt = x.transpose(1, 0, 2).reshape(N_IN, D)
    idx_t = connection_indices.astype(jnp.int32).T
    w_big = (
        jnp.eye(B, dtype=jnp.float32)[None, :, None, :, None]
        * kernel[:, None, :, None, :]
    ).reshape(KW, NF)
    w_t = w_big.T
    ws = (w_t[:, :D], w_t[:, D:2 * D], w_t[:, 2 * D:3 * D], w_t[:, 3 * D:])
    bias_t = jnp.tile(bias, B).reshape(NF, 1)

    zs = [
        _sc_gather(xt, idx_t[:, s * P_S:(s + 1) * P_S].reshape(NW, IDX_ROWS, IDX_MINOR))
        for s in range(NSLICE)
    ]
    y = _matmuls[0](zs[0], zs[0], zs[0], zs[0], *ws, bias_t)
    for s in range(1, NSLICE):
        y = _matmuls[s](zs[s], zs[s], zs[s], zs[s], *ws, bias_t, y)
    return y.reshape(B, FILTERS, N_OUT).transpose(0, 2, 1)


# speedup vs baseline: 1.2935x; 1.0266x over previous
import functools

import jax
import jax.numpy as jnp
from jax import lax
from jax.experimental import pallas as pl
from jax.experimental.pallas import tpu as pltpu
from jax.experimental.pallas import tpu_sc as plsc

B = 8
N_IN = 196608
N_OUT = 49152
K = 4
C_IN = 16
FILTERS = 32
NK = N_OUT * K
D = B * C_IN

NC = 2
NS = 16
NW = NC * NS
PER_W = NK // NW
IDX_MINOR = 128
IDX_ROWS = PER_W // IDX_MINOR
CHUNK_ROWS = 2
CHUNK = CHUNK_ROWS * IDX_MINOR
NCHUNK = IDX_ROWS // CHUNK_ROWS

_sc_mesh = plsc.VectorSubcoreMesh(core_axis_name="c", subcore_axis_name="s")


NBUF = 3
NROUND = NCHUNK // NBUF


@functools.partial(
    pl.kernel,
    out_type=jax.ShapeDtypeStruct((NK, D), jnp.float32),
    mesh=_sc_mesh,
    scratch_types=[
        pltpu.VMEM((IDX_ROWS, IDX_MINOR), jnp.int32),
        pltpu.VMEM((NBUF, CHUNK, D), jnp.float32),
        pltpu.SemaphoreType.DMA,
        pltpu.SemaphoreType.DMA,
        pltpu.SemaphoreType.DMA,
        pltpu.SemaphoreType.DMA,
        pltpu.SemaphoreType.DMA,
        pltpu.SemaphoreType.DMA,
    ],
)
def _sc_gather(xt_hbm, idx_hbm, z_hbm, idx_v, rows_v, g0, g1, g2, w0, w1, w2):
    gsem = (g0, g1, g2)
    wsem = (w0, w1, w2)
    wid = lax.axis_index("s") * NC + lax.axis_index("c")
    base = wid * PER_W
    pltpu.sync_copy(idx_hbm.at[wid], idx_v)

    def issue_gather(ch, p):
        for j in range(CHUNK_ROWS):
            pltpu.async_copy(
                xt_hbm.at[idx_v.at[ch * CHUNK_ROWS + j]],
                rows_v.at[p].at[pl.ds(j * IDX_MINOR, IDX_MINOR)],
                gsem[p],
            )

    def wait_gather(p):
        pltpu.make_async_copy(z_hbm.at[pl.ds(0, CHUNK)], rows_v.at[p], gsem[p]).wait()

    def issue_wb(ch, p):
        pltpu.async_copy(rows_v.at[p], z_hbm.at[pl.ds(base + ch * CHUNK, CHUNK)], wsem[p])

    def wait_wb(p):
        pltpu.make_async_copy(z_hbm.at[pl.ds(0, CHUNK)], rows_v.at[p], wsem[p]).wait()

    issue_gather(0, 0)
    issue_gather(1, 1)

    def round_(r, carry):
        for p in range(NBUF):
            ch = r * NBUF + p
            if p == 0:
                @pl.when(r >= 1)
                def _():
                    wait_wb(2)
                issue_gather(ch + 2, 2)
            else:
                @pl.when(ch + 2 < NCHUNK)
                def _():
                    wait_wb(p - 1)
                    issue_gather(ch + 2, p - 1)
            wait_gather(p)
            issue_wb(ch, p)
        return carry

    lax.fori_loop(0, NROUND, round_, 0)
    for p in range(NBUF):
        wait_wb(p)


TILE = 8192
KW = K * D
NF = B * FILTERS


NT = N_OUT // TILE


def _mm_body(z0, z1, z2, z3, w0, w1, w2, w3, bias_ref, o_ref):
    acc = bias_ref[...]
    for zk, wk in zip((z0, z1, z2, z3), (w0, w1, w2, w3)):
        acc = acc + jax.lax.dot_general(
            wk[...], zk[...],
            dimension_numbers=(((1,), (1,)), ((), ())),
            preferred_element_type=jnp.float32,
        )
    o_ref[...] = acc


def _z_spec(k):
    return pl.BlockSpec((TILE, D), lambda i, k=k: (k * NT + i, 0))


_matmul = pl.pallas_call(
    _mm_body,
    grid=(NT,),
    in_specs=[
        _z_spec(0), _z_spec(1), _z_spec(2), _z_spec(3),
        pl.BlockSpec((NF, D), lambda i: (0, 0)),
        pl.BlockSpec((NF, D), lambda i: (0, 0)),
        pl.BlockSpec((NF, D), lambda i: (0, 0)),
        pl.BlockSpec((NF, D), lambda i: (0, 0)),
        pl.BlockSpec((NF, 1), lambda i: (0, 0)),
    ],
    out_specs=pl.BlockSpec((NF, TILE), lambda i: (0, i)),
    out_shape=jax.ShapeDtypeStruct((NF, N_OUT), jnp.float32),
)


def kernel(x, connection_indices, kernel, bias):
    xt = x.transpose(1, 0, 2).reshape(N_IN, D)
    idx = connection_indices.astype(jnp.int32).T.reshape(NW, IDX_ROWS, IDX_MINOR)
    z = _sc_gather(xt, idx)
    w_big = (
        jnp.eye(B, dtype=jnp.float32)[None, :, None, :, None]
        * kernel[:, None, :, None, :]
    ).reshape(KW, NF)
    bias_t = jnp.tile(bias, B).reshape(NF, 1)
    w_t = w_big.T
    y = _matmul(z, z, z, z, w_t[:, :D], w_t[:, D:2 * D],
                w_t[:, 2 * D:3 * D], w_t[:, 3 * D:], bias_t)
    return y.reshape(B, FILTERS, N_OUT).transpose(0, 2, 1)
